# baseline (speedup 1.0000x reference)
import jax
import jax.numpy as jnp
from jax.experimental import pallas as pl


def _rup(n, m=128):
    return ((n + m - 1) // m) * m


def _pad2(a, r, c):
    return jnp.pad(a, ((0, r - a.shape[0]), (0, c - a.shape[1])))


def _mm_stats_kernel(x_ref, w_ref, b_ref, y_ref, st_ref):
    i = pl.program_id(0)
    y = jnp.dot(x_ref[...], w_ref[...], preferred_element_type=jnp.float32)
    y = y + b_ref[...]
    y_ref[...] = y
    tm = y.shape[0]
    mu = jnp.mean(y, axis=0, keepdims=True)
    d = y - mu
    m2 = jnp.sum(d * d, axis=0, keepdims=True)

    @pl.when(i == 0)
    def _():
        st_ref[...] = jnp.concatenate([mu, m2], axis=0)

    @pl.when(i > 0)
    def _():
        prev = st_ref[...]
        mean_a = prev[0:1, :]
        m2_a = prev[1:2, :]
        na = (i * tm).astype(jnp.float32)
        nb = jnp.float32(tm)
        n = na + nb
        delta = mu - mean_a
        mean_n = mean_a + delta * (nb / n)
        m2_n = m2_a + m2 + delta * delta * (na * nb / n)
        st_ref[...] = jnp.concatenate([mean_n, m2_n], axis=0)


def _mm_stats(x, wt, b):
    m, cin = x.shape
    cout = wt.shape[1]
    tm = m if m <= 1024 else 1024
    grid = m // tm
    y, st = pl.pallas_call(
        _mm_stats_kernel,
        grid=(grid,),
        in_specs=[
            pl.BlockSpec((tm, cin), lambda i: (i, 0)),
            pl.BlockSpec((cin, cout), lambda i: (0, 0)),
            pl.BlockSpec((1, cout), lambda i: (0, 0)),
        ],
        out_specs=[
            pl.BlockSpec((tm, cout), lambda i: (i, 0)),
            pl.BlockSpec((2, cout), lambda i: (0, 0)),
        ],
        out_shape=[
            jax.ShapeDtypeStruct((m, cout), jnp.float32),
            jax.ShapeDtypeStruct((2, cout), jnp.float32),
        ],
    )(x, wt, b)
    return y, st


def _arelu_kernel(y_ref, sc_ref, sh_ref, o_ref):
    o_ref[...] = jnp.maximum(y_ref[...] * sc_ref[...] + sh_ref[...], 0.0)


def _arelu(y, sc, sh):
    m, c = y.shape
    tm = m if m <= 1024 else 1024
    grid = m // tm
    return pl.pallas_call(
        _arelu_kernel,
        grid=(grid,),
        in_specs=[
            pl.BlockSpec((tm, c), lambda i: (i, 0)),
            pl.BlockSpec((1, c), lambda i: (0, 0)),
            pl.BlockSpec((1, c), lambda i: (0, 0)),
        ],
        out_specs=pl.BlockSpec((tm, c), lambda i: (i, 0)),
        out_shape=jax.ShapeDtypeStruct((m, c), jnp.float32),
    )(y, sc, sh)


def _arelu_pool_kernel(y_ref, sc_ref, sh_ref, o_ref, *, k):
    a = jnp.maximum(y_ref[...] * sc_ref[...] + sh_ref[...], 0.0)
    ts = a.shape[0] // k
    o_ref[...] = jnp.max(a.reshape(ts, k, a.shape[1]), axis=1)


import functools


def _arelu_pool(y, sc, sh, k):
    m, c = y.shape
    s = m // k
    ts = min(s, max(1, 4096 // k))
    grid = s // ts
    return pl.pallas_call(
        functools.partial(_arelu_pool_kernel, k=k),
        grid=(grid,),
        in_specs=[
            pl.BlockSpec((ts * k, c), lambda i: (i, 0)),
            pl.BlockSpec((1, c), lambda i: (0, 0)),
            pl.BlockSpec((1, c), lambda i: (0, 0)),
        ],
        out_specs=pl.BlockSpec((ts, c), lambda i: (i, 0)),
        out_shape=jax.ShapeDtypeStruct((s, c), jnp.float32),
    )(y, sc, sh)


def _scale_shift(st, m_rows, g, be, cout):
    cp = _rup(cout)
    mean = st[0:1, :]
    var = st[1:2, :] / jnp.float32(m_rows)
    gp = _pad2(g.reshape(1, -1), 1, cp)
    bep = _pad2(be.reshape(1, -1), 1, cp)
    sc = gp * jax.lax.rsqrt(var + 1e-5)
    sh = bep - mean * sc
    return sc, sh


def _conv_chain(x, layers):
    m = x.shape[0]
    cp = _rup(x.shape[1])
    xp = _pad2(x, m, cp)
    y = sc = sh = None
    ct = x.shape[1]
    for li, (w, b, g, be) in enumerate(layers):
        cout = w.shape[0]
        cop = _rup(cout)
        wt = _pad2(w.T, cp, cop)
        bp = _pad2(b.reshape(1, -1), 1, cop)
        if li > 0:
            xp = _arelu(y, sc, sh)
        y, st = _mm_stats(xp, wt, bp)
        sc, sh = _scale_shift(st, m, g, be, cout)
        cp = cop
        ct = cout
    return y, sc, sh, ct


def _square_distance(src, dst):
    return (jnp.sum(src ** 2, -1)[:, :, None]
            + jnp.sum(dst ** 2, -1)[:, None, :]
            - 2.0 * jnp.einsum('bnc,bmc->bnm', src, dst))


def _index_points(points, idx):
    return jax.vmap(lambda p, i: p[i])(points, idx)


def _fps(xyz, npoint):
    xyz = jax.lax.stop_gradient(xyz)
    B, N, _ = xyz.shape

    def step(state, _):
        distance, farthest = state
        centroid = jax.vmap(lambda p, i: p[i])(xyz, farthest)[:, None, :]
        d = jnp.sum((xyz - centroid) ** 2, -1)
        distance = jnp.minimum(distance, d)
        nxt = jnp.argmax(distance, -1).astype(jnp.int32)
        return (distance, nxt), farthest

    init = (jnp.full((B, N), 1e10, dtype=xyz.dtype), jnp.zeros((B,), jnp.int32))
    _, idxs = jax.lax.scan(step, init, None, length=npoint)
    return jnp.transpose(idxs, (1, 0))


def _query_ball(radius, nsample, xyz, new_xyz):
    B, N, _ = xyz.shape
    S = new_xyz.shape[1]
    sqrdists = _square_distance(new_xyz, xyz)
    base = jnp.broadcast_to(jnp.arange(N, dtype=jnp.int32), (B, S, N))
    gi = jnp.where(sqrdists > radius ** 2, N, base)
    gi = -jax.lax.top_k(-gi, nsample)[0]
    first = jnp.broadcast_to(gi[:, :, :1], gi.shape)
    return jnp.where(gi == N, first, gi)


def _sa_msg(xyz, feats, npoint, radii, ks, branch_params):
    fps_idx = _fps(xyz, npoint)
    new_xyz = _index_points(xyz, fps_idx)
    outs = []
    for radius, K, bp in zip(radii, ks, branch_params):
        gi = _query_ball(radius, K, xyz, new_xyz)
        gx = _index_points(xyz, gi) - new_xyz[:, :, None, :]
        if feats is not None:
            g = jnp.concatenate([_index_points(feats, gi), gx], axis=-1)
        else:
            g = gx
        B, S, Kk, C = g.shape
        y, sc, sh, ct = _conv_chain(g.reshape(B * S * Kk, C), bp)
        pooled = _arelu_pool(y, sc, sh, Kk)[:, :ct]
        outs.append(pooled.reshape(B, S, ct))
    return new_xyz, jnp.concatenate(outs, axis=-1)


def _sa_group_all(xyz, feats, layer_params):
    B, N, C = xyz.shape
    new_xyz = jnp.zeros((B, 1, C), dtype=xyz.dtype)
    g = xyz if feats is None else jnp.concatenate([xyz, feats], axis=-1)
    y, sc, sh, ct = _conv_chain(g.reshape(B * N, g.shape[-1]), layer_params)
    pooled = _arelu_pool(y, sc, sh, N)[:, :ct]
    return new_xyz, pooled.reshape(B, 1, ct)


def _fp(xyz1, xyz2, points1, points2, layer_params):
    B, N, _ = xyz1.shape
    S = xyz2.shape[1]
    if S == 1:
        interp = jnp.broadcast_to(points2, (B, N, points2.shape[-1]))
    else:
        dists = _square_distance(xyz1, xyz2)
        negd, idx = jax.lax.top_k(-dists, 3)
        d = jnp.maximum(-negd, 0.0)
        recip = 1.0 / (d + 1e-8)
        w = recip / jnp.sum(recip, axis=2, keepdims=True)
        interp = jnp.sum(_index_points(points2, idx) * w[..., None], axis=2)
    x = jnp.concatenate([points1, interp], axis=-1) if points1 is not None else interp
    y, sc, sh, ct = _conv_chain(x.reshape(B * N, x.shape[-1]), layer_params)
    out = _arelu(y, sc, sh)[:, :ct]
    return out.reshape(B, N, ct)


def kernel(points, params):
    l0_xyz = points
    l1_xyz, l1_points = _sa_msg(l0_xyz, None, 1024, [0.02, 0.04, 0.08], [32, 64, 128], params['sa1'])
    l2_xyz, l2_points = _sa_msg(l1_xyz, l1_points, 512, [0.04, 0.08, 0.16], [64, 64, 128], params['sa2'])
    l3_xyz, l3_points = _sa_msg(l2_xyz, l2_points, 128, [0.08, 0.16, 0.32], [64, 64, 128], params['sa3'])
    l4_xyz, l4_points = _sa_group_all(l3_xyz, l3_points, params['sa4'])
    l3p = _fp(l3_xyz, l4_xyz, l3_points, l4_points, params['fp3'])
    l2p = _fp(l2_xyz, l3_xyz, l2_points, l3p, params['fp2'])
    l1p = _fp(l1_xyz, l2_xyz, l1_points, l2p, params['fp1'])

    B, Np, _ = l1p.shape
    y, sc, sh, ct = _conv_chain(l1p.reshape(B * Np, l1p.shape[-1]), [params['head1']])
    x = _arelu(y, sc, sh)

    w2, b2 = params['head2']
    cp = x.shape[1]
    wt2 = _pad2(w2.T, cp, 128)
    b2p = _pad2(b2.reshape(1, -1), 1, 128)
    out, _ = _mm_stats(x, wt2, b2p)
    return out[:, :1].reshape(B, Np, 1)


# speedup vs baseline: 1.3014x; 1.3014x over previous
import functools

import jax
import jax.numpy as jnp
from jax.experimental import pallas as pl
from jax.experimental.pallas import tpu as pltpu


def _rup(n, m=128):
    return ((n + m - 1) // m) * m


def _pad2(a, r, c):
    return jnp.pad(a, ((0, r - a.shape[0]), (0, c - a.shape[1])))


def _mm_stats_kernel(x_ref, w_ref, b_ref, y_ref, st_ref):
    i = pl.program_id(0)
    y = jnp.dot(x_ref[...], w_ref[...], preferred_element_type=jnp.float32)
    y = y + b_ref[...]
    y_ref[...] = y
    tm = y.shape[0]
    mu = jnp.mean(y, axis=0, keepdims=True)
    d = y - mu
    m2 = jnp.sum(d * d, axis=0, keepdims=True)

    @pl.when(i == 0)
    def _():
        st_ref[...] = jnp.concatenate([mu, m2], axis=0)

    @pl.when(i > 0)
    def _():
        prev = st_ref[...]
        mean_a = prev[0:1, :]
        m2_a = prev[1:2, :]
        na = (i * tm).astype(jnp.float32)
        nb = jnp.float32(tm)
        n = na + nb
        delta = mu - mean_a
        mean_n = mean_a + delta * (nb / n)
        m2_n = m2_a + m2 + delta * delta * (na * nb / n)
        st_ref[...] = jnp.concatenate([mean_n, m2_n], axis=0)


def _mm_stats(x, wt, b):
    m, cin = x.shape
    cout = wt.shape[1]
    tm = m if m <= 1024 else 1024
    grid = m // tm
    y, st = pl.pallas_call(
        _mm_stats_kernel,
        grid=(grid,),
        in_specs=[
            pl.BlockSpec((tm, cin), lambda i: (i, 0)),
            pl.BlockSpec((cin, cout), lambda i: (0, 0)),
            pl.BlockSpec((1, cout), lambda i: (0, 0)),
        ],
        out_specs=[
            pl.BlockSpec((tm, cout), lambda i: (i, 0)),
            pl.BlockSpec((2, cout), lambda i: (0, 0)),
        ],
        out_shape=[
            jax.ShapeDtypeStruct((m, cout), jnp.float32),
            jax.ShapeDtypeStruct((2, cout), jnp.float32),
        ],
    )(x, wt, b)
    return y, st


def _arelu_kernel(y_ref, sc_ref, sh_ref, o_ref):
    o_ref[...] = jnp.maximum(y_ref[...] * sc_ref[...] + sh_ref[...], 0.0)


def _arelu(y, sc, sh):
    m, c = y.shape
    tm = m if m <= 1024 else 1024
    grid = m // tm
    return pl.pallas_call(
        _arelu_kernel,
        grid=(grid,),
        in_specs=[
            pl.BlockSpec((tm, c), lambda i: (i, 0)),
            pl.BlockSpec((1, c), lambda i: (0, 0)),
            pl.BlockSpec((1, c), lambda i: (0, 0)),
        ],
        out_specs=pl.BlockSpec((tm, c), lambda i: (i, 0)),
        out_shape=jax.ShapeDtypeStruct((m, c), jnp.float32),
    )(y, sc, sh)


def _arelu_pool_kernel(y_ref, sc_ref, sh_ref, o_ref, *, k):
    a = jnp.maximum(y_ref[...] * sc_ref[...] + sh_ref[...], 0.0)
    ts = a.shape[0] // k
    o_ref[...] = jnp.max(a.reshape(ts, k, a.shape[1]), axis=1)


def _arelu_pool(y, sc, sh, k):
    m, c = y.shape
    s = m // k
    ts = min(s, max(1, 4096 // k))
    grid = s // ts
    return pl.pallas_call(
        functools.partial(_arelu_pool_kernel, k=k),
        grid=(grid,),
        in_specs=[
            pl.BlockSpec((ts * k, c), lambda i: (i, 0)),
            pl.BlockSpec((1, c), lambda i: (0, 0)),
            pl.BlockSpec((1, c), lambda i: (0, 0)),
        ],
        out_specs=pl.BlockSpec((ts, c), lambda i: (i, 0)),
        out_shape=jax.ShapeDtypeStruct((s, c), jnp.float32),
    )(y, sc, sh)


def _scale_shift(st, m_rows, g, be, cout):
    cp = _rup(cout)
    mean = st[0:1, :]
    var = st[1:2, :] / jnp.float32(m_rows)
    gp = _pad2(g.reshape(1, -1), 1, cp)
    bep = _pad2(be.reshape(1, -1), 1, cp)
    sc = gp * jax.lax.rsqrt(var + 1e-5)
    sh = bep - mean * sc
    return sc, sh


def _conv_chain(x, layers):
    m = x.shape[0]
    cp = _rup(x.shape[1])
    xp = _pad2(x, m, cp)
    y = sc = sh = None
    ct = x.shape[1]
    for li, (w, b, g, be) in enumerate(layers):
        cout = w.shape[0]
        cop = _rup(cout)
        wt = _pad2(w.T, cp, cop)
        bp = _pad2(b.reshape(1, -1), 1, cop)
        if li > 0:
            xp = _arelu(y, sc, sh)
        y, st = _mm_stats(xp, wt, bp)
        sc, sh = _scale_shift(st, m, g, be, cout)
        cp = cop
        ct = cout
    return y, sc, sh, ct


def _square_distance(src, dst):
    return (jnp.sum(src ** 2, -1)[:, :, None]
            + jnp.sum(dst ** 2, -1)[:, None, :]
            - 2.0 * jnp.einsum('bnc,bmc->bnm', src, dst))


def _index_points(points, idx):
    return jax.vmap(lambda p, i: p[i])(points, idx)


def _fps_kernel(xt_ref, idx_ref, dist_scr, *, npoint):
    rows = xt_ref.shape[0] // 3
    cols = xt_ref.shape[1]
    n = rows * cols
    x = xt_ref[0:rows, :]
    y = xt_ref[rows:2 * rows, :]
    z = xt_ref[2 * rows:3 * rows, :]
    lin = (jax.lax.broadcasted_iota(jnp.int32, (rows, cols), 0) * cols
           + jax.lax.broadcasted_iota(jnp.int32, (rows, cols), 1))
    iota_np = jax.lax.broadcasted_iota(jnp.int32, (1, npoint), 1)
    dist_scr[...] = jnp.full((rows, cols), 1e10, jnp.float32)
    idx_ref[...] = jnp.zeros((1, npoint), jnp.int32)

    def body(i, far):
        sel = lin == far
        cx = jnp.sum(jnp.where(sel, x, 0.0))
        cy = jnp.sum(jnp.where(sel, y, 0.0))
        cz = jnp.sum(jnp.where(sel, z, 0.0))
        d = (x - cx) ** 2 + (y - cy) ** 2 + (z - cz) ** 2
        dist = jnp.minimum(dist_scr[...], d)
        dist_scr[...] = dist
        idx_ref[...] = jnp.where(iota_np == i, far, idx_ref[...])
        m = jnp.max(dist)
        nxt = jnp.min(jnp.where(dist >= m, lin, n))
        return nxt

    jax.lax.fori_loop(0, npoint, body, jnp.int32(0))


def _fps(xyz, npoint):
    B, N, _ = xyz.shape
    rows = 8
    cols = N // rows
    xt = jnp.concatenate([xyz[0, :, c].reshape(rows, cols) for c in range(3)], axis=0)
    idx = pl.pallas_call(
        functools.partial(_fps_kernel, npoint=npoint),
        grid=(1,),
        in_specs=[pl.BlockSpec((3 * rows, cols), lambda i: (0, 0))],
        out_specs=pl.BlockSpec((1, npoint), lambda i: (0, 0)),
        out_shape=jax.ShapeDtypeStruct((1, npoint), jnp.int32),
        scratch_shapes=[pltpu.VMEM((rows, cols), jnp.float32)],
    )(xt)
    return idx


def _query_ball(radius, nsample, xyz, new_xyz):
    B, N, _ = xyz.shape
    S = new_xyz.shape[1]
    sqrdists = _square_distance(new_xyz, xyz)
    base = jnp.broadcast_to(jnp.arange(N, dtype=jnp.int32), (B, S, N))
    gi = jnp.where(sqrdists > radius ** 2, N, base)
    gi = -jax.lax.top_k(-gi, nsample)[0]
    first = jnp.broadcast_to(gi[:, :, :1], gi.shape)
    return jnp.where(gi == N, first, gi)


def _sa_msg(xyz, feats, npoint, radii, ks, branch_params):
    fps_idx = _fps(xyz, npoint)
    new_xyz = _index_points(xyz, fps_idx)
    outs = []
    for radius, K, bp in zip(radii, ks, branch_params):
        gi = _query_ball(radius, K, xyz, new_xyz)
        gx = _index_points(xyz, gi) - new_xyz[:, :, None, :]
        if feats is not None:
            g = jnp.concatenate([_index_points(feats, gi), gx], axis=-1)
        else:
            g = gx
        B, S, Kk, C = g.shape
        y, sc, sh, ct = _conv_chain(g.reshape(B * S * Kk, C), bp)
        pooled = _arelu_pool(y, sc, sh, Kk)[:, :ct]
        outs.append(pooled.reshape(B, S, ct))
    return new_xyz, jnp.concatenate(outs, axis=-1)


def _sa_group_all(xyz, feats, layer_params):
    B, N, C = xyz.shape
    new_xyz = jnp.zeros((B, 1, C), dtype=xyz.dtype)
    g = xyz if feats is None else jnp.concatenate([xyz, feats], axis=-1)
    y, sc, sh, ct = _conv_chain(g.reshape(B * N, g.shape[-1]), layer_params)
    pooled = _arelu_pool(y, sc, sh, N)[:, :ct]
    return new_xyz, pooled.reshape(B, 1, ct)


def _fp(xyz1, xyz2, points1, points2, layer_params):
    B, N, _ = xyz1.shape
    S = xyz2.shape[1]
    if S == 1:
        interp = jnp.broadcast_to(points2, (B, N, points2.shape[-1]))
    else:
        dists = _square_distance(xyz1, xyz2)
        negd, idx = jax.lax.top_k(-dists, 3)
        d = jnp.maximum(-negd, 0.0)
        recip = 1.0 / (d + 1e-8)
        w = recip / jnp.sum(recip, axis=2, keepdims=True)
        interp = jnp.sum(_index_points(points2, idx) * w[..., None], axis=2)
    x = jnp.concatenate([points1, interp], axis=-1) if points1 is not None else interp
    y, sc, sh, ct = _conv_chain(x.reshape(B * N, x.shape[-1]), layer_params)
    out = _arelu(y, sc, sh)[:, :ct]
    return out.reshape(B, N, ct)


def kernel(points, params):
    l0_xyz = points
    l1_xyz, l1_points = _sa_msg(l0_xyz, None, 1024, [0.02, 0.04, 0.08], [32, 64, 128], params['sa1'])
    l2_xyz, l2_points = _sa_msg(l1_xyz, l1_points, 512, [0.04, 0.08, 0.16], [64, 64, 128], params['sa2'])
    l3_xyz, l3_points = _sa_msg(l2_xyz, l2_points, 128, [0.08, 0.16, 0.32], [64, 64, 128], params['sa3'])
    l4_xyz, l4_points = _sa_group_all(l3_xyz, l3_points, params['sa4'])
    l3p = _fp(l3_xyz, l4_xyz, l3_points, l4_points, params['fp3'])
    l2p = _fp(l2_xyz, l3_xyz, l2_points, l3p, params['fp2'])
    l1p = _fp(l1_xyz, l2_xyz, l1_points, l2p, params['fp1'])

    B, Np, _ = l1p.shape
    y, sc, sh, ct = _conv_chain(l1p.reshape(B * Np, l1p.shape[-1]), [params['head1']])
    x = _arelu(y, sc, sh)

    w2, b2 = params['head2']
    cp = x.shape[1]
    wt2 = _pad2(w2.T, cp, 128)
    b2p = _pad2(b2.reshape(1, -1), 1, 128)
    out, _ = _mm_stats(x, wt2, b2p)
    return out[:, :1].reshape(B, Np, 1)
